# baseline (speedup 1.0000x reference)
import functools

import jax
import jax.numpy as jnp
from jax import lax
from jax.experimental import pallas as pl
from jax.experimental.pallas import tpu as pltpu
from jax.experimental.pallas import tpu_sc as plsc

N_NODES = 10000
N_EDGES = 320000
D = 128
LN_EPS = 1e-5

SPLITS = (96000, 128000, 96000)
SC_WORKERS = 32
CHUNK = 40
N_SPLITS = len(SPLITS)
NBUF = 5
NBUF_G = 3
NBUF_S = 3
N_PAD = 10240

NB_NODE = 2000
NB_EDGE = 8000


def _ln(h, g, b):
    m = jnp.mean(h, axis=-1, keepdims=True)
    v = jnp.mean((h - m) ** 2, axis=-1, keepdims=True)
    return (h - m) * lax.rsqrt(v + LN_EPS) * g + b


def _relu(h):
    return jnp.maximum(h, 0.0)


def _full(shape):
    return pl.BlockSpec(shape, lambda i: tuple(0 for _ in shape))


def _rows(nb, d):
    return pl.BlockSpec((nb, d), lambda i: (i, 0))


def _enc_body(x, w1, b1, w2, b2, w3, b3, g, bb, out):
    h = _relu(x[...] @ w1[...] + b1[...])
    h = _relu(h @ w2[...] + b2[...])
    h = h @ w3[...] + b3[...]
    out[...] = _ln(h, g[...], bb[...])


def _enc_body_bf(x, w1, b1, w2, b2, w3, b3, g, bb, out, out_bf):
    h = _relu(x[...] @ w1[...] + b1[...])
    h = _relu(h @ w2[...] + b2[...])
    h = h @ w3[...] + b3[...]
    nn = _ln(h, g[...], bb[...])
    out[...] = nn
    out_bf[...] = nn.astype(jnp.bfloat16)


def _encode(arr, p, nb, with_bf16=False):
    n, din = arr.shape
    args = (arr, p["W"][0], p["b"][0].reshape(1, -1), p["W"][1],
            p["b"][1].reshape(1, -1), p["W"][2], p["b"][2].reshape(1, -1),
            p["ln_g"].reshape(1, -1), p["ln_b"].reshape(1, -1))
    in_specs = [_rows(nb, din), _full((din, D)), _full((1, D)),
                _full((D, D)), _full((1, D)), _full((D, D)), _full((1, D)),
                _full((1, D)), _full((1, D))]
    if with_bf16:
        return pl.pallas_call(
            _enc_body_bf,
            grid=(n // nb,),
            in_specs=in_specs,
            out_specs=[_rows(nb, D), _rows(nb, D)],
            out_shape=[jax.ShapeDtypeStruct((n, D), jnp.float32),
                       jax.ShapeDtypeStruct((n, D), jnp.bfloat16)],
        )(*args)
    return pl.pallas_call(
        _enc_body,
        grid=(n // nb,),
        in_specs=in_specs,
        out_specs=_rows(nb, D),
        out_shape=jax.ShapeDtypeStruct((n, D), jnp.float32),
    )(*args)


def _edge_body(e, gs, gd, w1e, w1s, w1d, b1, w2, b2, w3, b3, g, bb, eo):
    hs = jnp.dot(gs[...], w1s[...], preferred_element_type=jnp.float32)
    hd = jnp.dot(gd[...], w1d[...], preferred_element_type=jnp.float32)
    h = _relu(e[...] @ w1e[...] + hs + hd + b1[...])
    h = _relu(h @ w2[...] + b2[...])
    h = h @ w3[...] + b3[...]
    eo[...] = e[...] + _ln(h, g[...], bb[...])


def _edge_step(edge, gs, gd, p):
    w1 = p["W"][0]
    args = (edge, gs, gd, w1[:D], w1[D:2 * D], w1[2 * D:],
            p["b"][0].reshape(1, -1), p["W"][1], p["b"][1].reshape(1, -1),
            p["W"][2], p["b"][2].reshape(1, -1),
            p["ln_g"].reshape(1, -1), p["ln_b"].reshape(1, -1))
    n = edge.shape[0]
    return pl.pallas_call(
        _edge_body,
        grid=(n // NB_EDGE,),
        in_specs=[_rows(NB_EDGE, D)] * 3
                 + [_full((D, D))] * 3 + [_full((1, D)), _full((D, D)),
                                          _full((1, D)), _full((D, D)),
                                          _full((1, D)), _full((1, D)),
                                          _full((1, D))],
        out_specs=_rows(NB_EDGE, D),
        out_shape=jax.ShapeDtypeStruct((n, D), jnp.float32),
    )(*args)


def _node_body(nd, *rest):
    aggs = rest[:N_SPLITS]
    (w1a, w1b, b1, w2, b2, w3, b3, g, bb, no) = rest[N_SPLITS:]
    a = sum(ag[0] + ag[1] for ag in aggs)
    h = _relu(nd[...] @ w1a[...] + a @ w1b[...] + b1[...])
    h = _relu(h @ w2[...] + b2[...])
    h = h @ w3[...] + b3[...]
    no[...] = nd[...] + _ln(h, g[...], bb[...])


def _node_step(node, aggs, p):
    w1 = p["W"][0]
    args = (node, *aggs, w1[:D], w1[D:], p["b"][0].reshape(1, -1),
            p["W"][1], p["b"][1].reshape(1, -1), p["W"][2],
            p["b"][2].reshape(1, -1), p["ln_g"].reshape(1, -1),
            p["ln_b"].reshape(1, -1))
    agg_spec = pl.BlockSpec((2, NB_NODE, D), lambda i: (0, i, 0))
    return pl.pallas_call(
        _node_body,
        grid=(N_NODES // NB_NODE,),
        in_specs=[_rows(NB_NODE, D)] + [agg_spec] * N_SPLITS
                 + [_full((D, D))] * 2 + [_full((1, D)), _full((D, D)),
                                          _full((1, D)), _full((D, D)),
                                          _full((1, D)), _full((1, D)),
                                          _full((1, D))],
        out_specs=_rows(NB_NODE, D),
        out_shape=jax.ShapeDtypeStruct((N_NODES, D), jnp.float32),
    )(*args)


def _node_dec_body(nd, *rest):
    aggs = rest[:N_SPLITS]
    (w1a, w1b, b1, w2, b2, w3, b3, g, bb,
     dw1, db1, dw2, db2, dw3, db3, oo) = rest[N_SPLITS:]
    a = sum(ag[0] + ag[1] for ag in aggs)
    h = _relu(nd[...] @ w1a[...] + a @ w1b[...] + b1[...])
    h = _relu(h @ w2[...] + b2[...])
    h = h @ w3[...] + b3[...]
    nn = nd[...] + _ln(h, g[...], bb[...])
    d = _relu(nn @ dw1[...] + db1[...])
    d = _relu(d @ dw2[...] + db2[...])
    oo[...] = d @ dw3[...] + db3[...]


def _node_step_decode(node, aggs, p, pdec):
    w1 = p["W"][0]
    dw3 = jnp.zeros((D, D), jnp.float32).at[:, :3].set(pdec["W"][2])
    db3 = jnp.zeros((1, D), jnp.float32).at[0, :3].set(pdec["b"][2])
    args = (node, *aggs, w1[:D], w1[D:], p["b"][0].reshape(1, -1),
            p["W"][1], p["b"][1].reshape(1, -1), p["W"][2],
            p["b"][2].reshape(1, -1), p["ln_g"].reshape(1, -1),
            p["ln_b"].reshape(1, -1),
            pdec["W"][0], pdec["b"][0].reshape(1, -1),
            pdec["W"][1], pdec["b"][1].reshape(1, -1), dw3, db3)
    agg_spec = pl.BlockSpec((2, NB_NODE, D), lambda i: (0, i, 0))
    return pl.pallas_call(
        _node_dec_body,
        grid=(N_NODES // NB_NODE,),
        in_specs=[_rows(NB_NODE, D)] + [agg_spec] * N_SPLITS
                 + [_full((D, D))] * 2 + [_full((1, D)), _full((D, D)),
                                          _full((1, D)), _full((D, D)),
                                          _full((1, D)), _full((1, D)),
                                          _full((1, D))]
                 + [_full((D, D)), _full((1, D)), _full((D, D)),
                    _full((1, D)), _full((D, D)), _full((1, D))],
        out_specs=_rows(NB_NODE, D),
        out_shape=jax.ShapeDtypeStruct((N_NODES, D), jnp.float32),
    )(*args)


@functools.lru_cache(maxsize=None)
def _gather_kernel(epw):
    mesh = plsc.VectorSubcoreMesh(core_axis_name="c", subcore_axis_name="s")
    n_half = epw * SC_WORKERS
    n_chunks = epw // CHUNK
    grp = n_chunks // NBUF_G

    def body(node_hbm, si_hbm, di_hbm, gs_hbm, gd_hbm,
             table_sh, si_v, di_v, b0, b1, b2, *sems):
        bufs = (b0, b1, b2)
        gsem = sems[:NBUF_G]
        wsem = sems[NBUF_G:]
        sid = lax.axis_index("s")
        wid = sid * 2 + lax.axis_index("c")
        pltpu.sync_copy(si_hbm.at[wid], si_v)
        pltpu.sync_copy(di_hbm.at[wid], di_v)

        @pl.when(sid == 0)
        def _():
            pltpu.sync_copy(node_hbm, table_sh)

        plsc.subcore_barrier()
        ebase = wid * epw

        def run(idx_v, out_hbm):
            def start_g(ch, b):
                pltpu.async_copy(table_sh.at[idx_v.at[ch]], bufs[b], gsem[b])

            def wait_g(ch, b):
                pltpu.make_async_copy(table_sh.at[idx_v.at[ch]], bufs[b],
                                      gsem[b]).wait()

            def start_w(ch, b):
                pltpu.async_copy(
                    bufs[b], out_hbm.at[pl.ds(ebase + ch * CHUNK, CHUNK)],
                    wsem[b])

            def wait_w(ch, b):
                pltpu.make_async_copy(
                    bufs[b], out_hbm.at[pl.ds(ebase + ch * CHUNK, CHUNK)],
                    wsem[b]).wait()

            for b in range(NBUF_G):
                start_g(b, b)
            for b in range(NBUF_G):
                wait_g(b, b)
                start_w(b, b)

            def round_body(r, carry):
                for b in range(NBUF_G):
                    ch = r * NBUF_G + b
                    wait_w(ch - NBUF_G, b)
                    start_g(ch, b)
                for b in range(NBUF_G):
                    ch = r * NBUF_G + b
                    wait_g(ch, b)
                    start_w(ch, b)
                return carry

            lax.fori_loop(1, grp, round_body, 0)
            n_done = grp * NBUF_G
            for b in range(n_chunks - n_done):
                ch = n_done + b
                wait_w(ch - NBUF_G, b)
                start_g(ch, b)
            for b in range(n_chunks - n_done):
                ch = n_done + b
                wait_g(ch, b)
                start_w(ch, b)
            for b in range(n_chunks - n_done, NBUF_G):
                wait_w(n_done - NBUF_G + b, b)
            for b in range(n_chunks - n_done):
                wait_w(n_done + b, b)

        run(si_v, gs_hbm)
        run(di_v, gd_hbm)

    return pl.kernel(
        body,
        out_type=(jax.ShapeDtypeStruct((n_half, D), jnp.float32),
                  jax.ShapeDtypeStruct((n_half, D), jnp.float32)),
        mesh=mesh,
        scratch_types=(
            [pltpu.VMEM_SHARED((N_NODES, D), jnp.float32)]
            + [pltpu.VMEM((n_chunks, CHUNK), jnp.int32)] * 2
            + [pltpu.VMEM((CHUNK, D), jnp.float32)] * NBUF_G
            + [pltpu.SemaphoreType.DMA] * (2 * NBUF_G)),
    )


def _gather_pair(node, si3d, di3d):
    return _gather_kernel(si3d.shape[1] * CHUNK)(node, si3d, di3d)


@functools.lru_cache(maxsize=None)
def _scatter_kernel(epw):
    mesh = plsc.VectorSubcoreMesh(core_axis_name="c", subcore_axis_name="s")
    rows_per_sub = N_PAD // 16
    n_chunks = epw // CHUNK
    n_grp = n_chunks // NBUF_S

    def body(e_hbm, di_hbm, z_hbm, agg_hbm, shared, di_v, *rest):
        bufs = rest[:NBUF_S]
        sems = rest[NBUF_S:]
        cid = lax.axis_index("c")
        sid = lax.axis_index("s")
        wid = sid * 2 + cid
        pltpu.sync_copy(di_hbm.at[wid], di_v)

        @pl.when(sid == 0)
        def _():
            pltpu.sync_copy(z_hbm, shared)

        plsc.subcore_barrier()

        ebase = wid * epw

        def start_load(ch, k):
            pltpu.async_copy(e_hbm.at[pl.ds(ebase + ch * CHUNK, CHUNK)],
                             bufs[k], sems[k])

        def wait_load(ch, k):
            pltpu.make_async_copy(
                e_hbm.at[pl.ds(ebase + ch * CHUNK, CHUNK)], bufs[k],
                sems[k]).wait()

        for b in range(NBUF_S):
            start_load(b, b)

        def grp_body(r, carry):
            for b in range(NBUF_S):
                ch = r * NBUF_S + b
                wait_load(ch, b)
                pltpu.sync_copy(bufs[b], shared.at[di_v.at[ch]], add=True)

                @pl.when(ch + NBUF_S < n_chunks)
                def _():
                    start_load(ch + NBUF_S, b)

            return carry

        lax.fori_loop(0, n_grp, grp_body, 0)
        for j in range(n_chunks - n_grp * NBUF_S):
            ch = n_grp * NBUF_S + j
            wait_load(ch, j)
            pltpu.sync_copy(bufs[j], shared.at[di_v.at[ch]], add=True)
        plsc.subcore_barrier()
        out_row = cid * N_PAD + sid * rows_per_sub
        pltpu.sync_copy(shared.at[pl.ds(sid * rows_per_sub, rows_per_sub)],
                        agg_hbm.at[pl.ds(out_row, rows_per_sub)])

    return pl.kernel(
        body,
        out_type=jax.ShapeDtypeStruct((2 * N_PAD, D), jnp.float32),
        mesh=mesh,
        scratch_types=(
            [pltpu.VMEM_SHARED((N_PAD, D), jnp.float32),
             pltpu.VMEM((n_chunks, CHUNK), jnp.int32)]
            + [pltpu.VMEM((CHUNK, D), jnp.float32)] * NBUF_S
            + [pltpu.SemaphoreType.DMA] * NBUF_S),
    )


def _scatter_partials(edge_new, di3d, zeros_nd):
    return _scatter_kernel(di3d.shape[1] * CHUNK)(edge_new, di3d, zeros_nd)


def kernel(x, edge_attr, params, edge_index):
    offs = [0]
    for s in SPLITS:
        offs.append(offs[-1] + s)
    si, di, edge = [], [], []
    for h, s in enumerate(SPLITS):
        lo, hi = offs[h], offs[h + 1]
        si.append(edge_index[0][lo:hi]
                  .reshape(SC_WORKERS, s // SC_WORKERS // CHUNK, CHUNK))
        di.append(edge_index[1][lo:hi]
                  .reshape(SC_WORKERS, s // SC_WORKERS // CHUNK, CHUNK))
    zeros_nd = jnp.zeros((N_PAD, D), jnp.float32)

    node = _encode(x, params["enc_node"], NB_NODE)
    for h, s in enumerate(SPLITS):
        edge.append(_encode(edge_attr[offs[h]:offs[h + 1]],
                            params["enc_edge"], NB_EDGE))

    n_sp = len(SPLITS)
    for t, blk in enumerate(params["proc"]):
        gpair = [None] * n_sp
        gpair[0] = _gather_pair(node, si[0], di[0])
        aggs = []
        new_edge = []
        for h in range(n_sp):
            if h + 1 < n_sp:
                gpair[h + 1] = _gather_pair(node, si[h + 1], di[h + 1])
            e_h = _edge_step(edge[h], gpair[h][0], gpair[h][1], blk["edge"])
            new_edge.append(e_h)
            aggs.append(_scatter_partials(e_h, di[h], zeros_nd)
                        .reshape(2, N_PAD, D))
        edge = new_edge
        if t < len(params["proc"]) - 1:
            node = _node_step(node, aggs, blk["node"])
        else:
            out = _node_step_decode(node, aggs, blk["node"], params["dec"])
    return out[:, :3]


# speedup vs baseline: 1.0271x; 1.0271x over previous
import functools

import jax
import jax.numpy as jnp
from jax import lax
from jax.experimental import pallas as pl
from jax.experimental.pallas import tpu as pltpu
from jax.experimental.pallas import tpu_sc as plsc

N_NODES = 10000
N_EDGES = 320000
D = 128
LN_EPS = 1e-5

SPLITS = (160000, 160000)
SC_WORKERS = 32
CHUNK = 40
N_SPLITS = len(SPLITS)
NBUF_G = 3
NBUF_S = 3
N_PAD = 10240

NB_NODE = 2000
NB_EDGE = 8000


def _ln(h, g, b):
    m = jnp.mean(h, axis=-1, keepdims=True)
    v = jnp.mean((h - m) ** 2, axis=-1, keepdims=True)
    return (h - m) * lax.rsqrt(v + LN_EPS) * g + b


def _relu(h):
    return jnp.maximum(h, 0.0)


def _full(shape):
    return pl.BlockSpec(shape, lambda i: tuple(0 for _ in shape))


def _rows(nb, d):
    return pl.BlockSpec((nb, d), lambda i: (i, 0))


def _enc_body(x, w1, b1, w2, b2, w3, b3, g, bb, out):
    h = _relu(x[...] @ w1[...] + b1[...])
    h = _relu(h @ w2[...] + b2[...])
    h = h @ w3[...] + b3[...]
    out[...] = _ln(h, g[...], bb[...])


def _encode(arr, p, nb):
    n, din = arr.shape
    args = (arr, p["W"][0], p["b"][0].reshape(1, -1), p["W"][1],
            p["b"][1].reshape(1, -1), p["W"][2], p["b"][2].reshape(1, -1),
            p["ln_g"].reshape(1, -1), p["ln_b"].reshape(1, -1))
    in_specs = [_rows(nb, din), _full((din, D)), _full((1, D)),
                _full((D, D)), _full((1, D)), _full((D, D)), _full((1, D)),
                _full((1, D)), _full((1, D))]
    return pl.pallas_call(
        _enc_body,
        grid=(n // nb,),
        in_specs=in_specs,
        out_specs=_rows(nb, D),
        out_shape=jax.ShapeDtypeStruct((n, D), jnp.float32),
    )(*args)


def _edge_body(e, gs, gd, w1e, w1s, w1d, b1, w2, b2, w3, b3, g, bb, eo):
    hs = jnp.dot(gs[...], w1s[...], preferred_element_type=jnp.float32)
    hd = jnp.dot(gd[...], w1d[...], preferred_element_type=jnp.float32)
    h = _relu(e[...] @ w1e[...] + hs + hd + b1[...])
    h = _relu(h @ w2[...] + b2[...])
    h = h @ w3[...] + b3[...]
    eo[...] = e[...] + _ln(h, g[...], bb[...])


def _edge_step(edge, gs, gd, p):
    w1 = p["W"][0]
    args = (edge, gs, gd, w1[:D], w1[D:2 * D], w1[2 * D:],
            p["b"][0].reshape(1, -1), p["W"][1], p["b"][1].reshape(1, -1),
            p["W"][2], p["b"][2].reshape(1, -1),
            p["ln_g"].reshape(1, -1), p["ln_b"].reshape(1, -1))
    n = edge.shape[0]
    return pl.pallas_call(
        _edge_body,
        grid=(n // NB_EDGE,),
        in_specs=[_rows(NB_EDGE, D)] * 3
                 + [_full((D, D))] * 3 + [_full((1, D)), _full((D, D)),
                                          _full((1, D)), _full((D, D)),
                                          _full((1, D)), _full((1, D)),
                                          _full((1, D))],
        out_specs=_rows(NB_EDGE, D),
        out_shape=jax.ShapeDtypeStruct((n, D), jnp.float32),
    )(*args)


def _node_body(nd, *rest):
    aggs = rest[:N_SPLITS]
    (w1a, w1b, b1, w2, b2, w3, b3, g, bb, no) = rest[N_SPLITS:]
    a = sum(ag[0] + ag[1] for ag in aggs)
    h = _relu(nd[...] @ w1a[...] + a @ w1b[...] + b1[...])
    h = _relu(h @ w2[...] + b2[...])
    h = h @ w3[...] + b3[...]
    no[...] = nd[...] + _ln(h, g[...], bb[...])


def _node_step(node, aggs, p):
    w1 = p["W"][0]
    args = (node, *aggs, w1[:D], w1[D:], p["b"][0].reshape(1, -1),
            p["W"][1], p["b"][1].reshape(1, -1), p["W"][2],
            p["b"][2].reshape(1, -1), p["ln_g"].reshape(1, -1),
            p["ln_b"].reshape(1, -1))
    agg_spec = pl.BlockSpec((2, NB_NODE, D), lambda i: (0, i, 0))
    return pl.pallas_call(
        _node_body,
        grid=(N_NODES // NB_NODE,),
        in_specs=[_rows(NB_NODE, D)] + [agg_spec] * N_SPLITS
                 + [_full((D, D))] * 2 + [_full((1, D)), _full((D, D)),
                                          _full((1, D)), _full((D, D)),
                                          _full((1, D)), _full((1, D)),
                                          _full((1, D))],
        out_specs=_rows(NB_NODE, D),
        out_shape=jax.ShapeDtypeStruct((N_NODES, D), jnp.float32),
    )(*args)


def _node_dec_body(nd, *rest):
    aggs = rest[:N_SPLITS]
    (w1a, w1b, b1, w2, b2, w3, b3, g, bb,
     dw1, db1, dw2, db2, dw3, db3, oo) = rest[N_SPLITS:]
    a = sum(ag[0] + ag[1] for ag in aggs)
    h = _relu(nd[...] @ w1a[...] + a @ w1b[...] + b1[...])
    h = _relu(h @ w2[...] + b2[...])
    h = h @ w3[...] + b3[...]
    nn = nd[...] + _ln(h, g[...], bb[...])
    d = _relu(nn @ dw1[...] + db1[...])
    d = _relu(d @ dw2[...] + db2[...])
    oo[...] = d @ dw3[...] + db3[...]


def _node_step_decode(node, aggs, p, pdec):
    w1 = p["W"][0]
    dw3 = jnp.zeros((D, D), jnp.float32).at[:, :3].set(pdec["W"][2])
    db3 = jnp.zeros((1, D), jnp.float32).at[0, :3].set(pdec["b"][2])
    args = (node, *aggs, w1[:D], w1[D:], p["b"][0].reshape(1, -1),
            p["W"][1], p["b"][1].reshape(1, -1), p["W"][2],
            p["b"][2].reshape(1, -1), p["ln_g"].reshape(1, -1),
            p["ln_b"].reshape(1, -1),
            pdec["W"][0], pdec["b"][0].reshape(1, -1),
            pdec["W"][1], pdec["b"][1].reshape(1, -1), dw3, db3)
    agg_spec = pl.BlockSpec((2, NB_NODE, D), lambda i: (0, i, 0))
    return pl.pallas_call(
        _node_dec_body,
        grid=(N_NODES // NB_NODE,),
        in_specs=[_rows(NB_NODE, D)] + [agg_spec] * N_SPLITS
                 + [_full((D, D))] * 2 + [_full((1, D)), _full((D, D)),
                                          _full((1, D)), _full((D, D)),
                                          _full((1, D)), _full((1, D)),
                                          _full((1, D))]
                 + [_full((D, D)), _full((1, D)), _full((D, D)),
                    _full((1, D)), _full((D, D)), _full((1, D))],
        out_specs=_rows(NB_NODE, D),
        out_shape=jax.ShapeDtypeStruct((N_NODES, D), jnp.float32),
    )(*args)


@functools.lru_cache(maxsize=None)
def _gather_kernel(epw):
    mesh = plsc.VectorSubcoreMesh(core_axis_name="c", subcore_axis_name="s")
    n_half = epw * SC_WORKERS
    n_chunks = epw // CHUNK
    grp = n_chunks // NBUF_G

    def body(node_hbm, si_hbm, di_hbm, gs_hbm, gd_hbm,
             table_sh, si_v, di_v, b0, b1, b2, *sems):
        bufs = (b0, b1, b2)
        gsem = sems[:NBUF_G]
        wsem = sems[NBUF_G:]
        sid = lax.axis_index("s")
        wid = sid * 2 + lax.axis_index("c")
        pltpu.sync_copy(si_hbm.at[wid], si_v)
        pltpu.sync_copy(di_hbm.at[wid], di_v)

        @pl.when(sid == 0)
        def _():
            pltpu.sync_copy(node_hbm, table_sh)

        plsc.subcore_barrier()
        ebase = wid * epw

        def run(idx_v, out_hbm):
            def start_g(ch, b):
                pltpu.async_copy(table_sh.at[idx_v.at[ch]], bufs[b], gsem[b])

            def wait_g(ch, b):
                pltpu.make_async_copy(table_sh.at[idx_v.at[ch]], bufs[b],
                                      gsem[b]).wait()

            def start_w(ch, b):
                pltpu.async_copy(
                    bufs[b], out_hbm.at[pl.ds(ebase + ch * CHUNK, CHUNK)],
                    wsem[b])

            def wait_w(ch, b):
                pltpu.make_async_copy(
                    bufs[b], out_hbm.at[pl.ds(ebase + ch * CHUNK, CHUNK)],
                    wsem[b]).wait()

            for b in range(NBUF_G):
                start_g(b, b)
            for b in range(NBUF_G):
                wait_g(b, b)
                start_w(b, b)

            def round_body(r, carry):
                for b in range(NBUF_G):
                    ch = r * NBUF_G + b
                    wait_w(ch - NBUF_G, b)
                    start_g(ch, b)
                for b in range(NBUF_G):
                    ch = r * NBUF_G + b
                    wait_g(ch, b)
                    start_w(ch, b)
                return carry

            lax.fori_loop(1, grp, round_body, 0)
            n_done = grp * NBUF_G
            for b in range(n_chunks - n_done):
                ch = n_done + b
                wait_w(ch - NBUF_G, b)
                start_g(ch, b)
            for b in range(n_chunks - n_done):
                ch = n_done + b
                wait_g(ch, b)
                start_w(ch, b)
            for b in range(n_chunks - n_done, NBUF_G):
                wait_w(n_done - NBUF_G + b, b)
            for b in range(n_chunks - n_done):
                wait_w(n_done + b, b)

        run(si_v, gs_hbm)
        run(di_v, gd_hbm)

    return pl.kernel(
        body,
        out_type=(jax.ShapeDtypeStruct((n_half, D), jnp.float32),
                  jax.ShapeDtypeStruct((n_half, D), jnp.float32)),
        mesh=mesh,
        scratch_types=(
            [pltpu.VMEM_SHARED((N_NODES, D), jnp.float32)]
            + [pltpu.VMEM((n_chunks, CHUNK), jnp.int32)] * 2
            + [pltpu.VMEM((CHUNK, D), jnp.float32)] * NBUF_G
            + [pltpu.SemaphoreType.DMA] * (2 * NBUF_G)),
    )


def _gather_pair(node, si3d, di3d):
    return _gather_kernel(si3d.shape[1] * CHUNK)(node, si3d, di3d)


@functools.lru_cache(maxsize=None)
def _scatter_kernel(epw):
    mesh = plsc.VectorSubcoreMesh(core_axis_name="c", subcore_axis_name="s")
    rows_per_sub = N_PAD // 16
    n_chunks = epw // CHUNK
    n_grp = n_chunks // NBUF_S

    def body(e_hbm, di_hbm, z_hbm, agg_hbm, shared, di_v, *rest):
        bufs = rest[:NBUF_S]
        sems = rest[NBUF_S:]
        cid = lax.axis_index("c")
        sid = lax.axis_index("s")
        wid = sid * 2 + cid
        pltpu.sync_copy(di_hbm.at[wid], di_v)

        @pl.when(sid == 0)
        def _():
            pltpu.sync_copy(z_hbm, shared)

        plsc.subcore_barrier()

        ebase = wid * epw

        def start_load(ch, k):
            pltpu.async_copy(e_hbm.at[pl.ds(ebase + ch * CHUNK, CHUNK)],
                             bufs[k], sems[k])

        def wait_load(ch, k):
            pltpu.make_async_copy(
                e_hbm.at[pl.ds(ebase + ch * CHUNK, CHUNK)], bufs[k],
                sems[k]).wait()

        for b in range(NBUF_S):
            start_load(b, b)

        def grp_body(r, carry):
            for b in range(NBUF_S):
                ch = r * NBUF_S + b
                wait_load(ch, b)
                pltpu.sync_copy(bufs[b], shared.at[di_v.at[ch]], add=True)

                @pl.when(ch + NBUF_S < n_chunks)
                def _():
                    start_load(ch + NBUF_S, b)

            return carry

        lax.fori_loop(0, n_grp, grp_body, 0)
        for j in range(n_chunks - n_grp * NBUF_S):
            ch = n_grp * NBUF_S + j
            wait_load(ch, j)
            pltpu.sync_copy(bufs[j], shared.at[di_v.at[ch]], add=True)
        plsc.subcore_barrier()
        out_row = cid * N_PAD + sid * rows_per_sub
        pltpu.sync_copy(shared.at[pl.ds(sid * rows_per_sub, rows_per_sub)],
                        agg_hbm.at[pl.ds(out_row, rows_per_sub)])

    return pl.kernel(
        body,
        out_type=jax.ShapeDtypeStruct((2 * N_PAD, D), jnp.float32),
        mesh=mesh,
        scratch_types=(
            [pltpu.VMEM_SHARED((N_PAD, D), jnp.float32),
             pltpu.VMEM((n_chunks, CHUNK), jnp.int32)]
            + [pltpu.VMEM((CHUNK, D), jnp.float32)] * NBUF_S
            + [pltpu.SemaphoreType.DMA] * NBUF_S),
    )


def _scatter_partials(edge_new, di3d, zeros_nd):
    return _scatter_kernel(di3d.shape[1] * CHUNK)(edge_new, di3d, zeros_nd)


def kernel(x, edge_attr, params, edge_index):
    offs = [0]
    for s in SPLITS:
        offs.append(offs[-1] + s)
    si, di, edge = [], [], []
    for h, s in enumerate(SPLITS):
        lo, hi = offs[h], offs[h + 1]
        si.append(edge_index[0][lo:hi]
                  .reshape(SC_WORKERS, s // SC_WORKERS // CHUNK, CHUNK))
        di.append(edge_index[1][lo:hi]
                  .reshape(SC_WORKERS, s // SC_WORKERS // CHUNK, CHUNK))
    zeros_nd = jnp.zeros((N_PAD, D), jnp.float32)

    node = _encode(x, params["enc_node"], NB_NODE)
    for h, s in enumerate(SPLITS):
        edge.append(_encode(edge_attr[offs[h]:offs[h + 1]],
                            params["enc_edge"], NB_EDGE))

    n_sp = len(SPLITS)
    for t, blk in enumerate(params["proc"]):
        gpair = [None] * n_sp
        gpair[0] = _gather_pair(node, si[0], di[0])
        aggs = []
        new_edge = []
        for h in range(n_sp):
            if h + 1 < n_sp:
                gpair[h + 1] = _gather_pair(node, si[h + 1], di[h + 1])
            e_h = _edge_step(edge[h], gpair[h][0], gpair[h][1], blk["edge"])
            new_edge.append(e_h)
            aggs.append(_scatter_partials(e_h, di[h], zeros_nd)
                        .reshape(2, N_PAD, D))
        edge = new_edge
        if t < len(params["proc"]) - 1:
            node = _node_step(node, aggs, blk["node"])
        else:
            out = _node_step_decode(node, aggs, blk["node"], params["dec"])
    return out[:, :3]
